# baseline (speedup 1.0000x reference)
import functools

import jax
import jax.numpy as jnp
from jax import lax
from jax.experimental import pallas as pl
from jax.experimental.pallas import tpu as pltpu
from jax.experimental.pallas import tpu_sc as plsc

HIDDEN = 64
BATCH = 16384
CHUNK = 128


@functools.cache
def _build(num_rows: int, batch: int, hidden: int):
    info = plsc.get_sparse_core_info()
    nw = info.num_cores * info.num_subcores
    b_per_w = batch // nw
    n_chunks = b_per_w // CHUNK

    mesh = plsc.VectorSubcoreMesh(core_axis_name="c", subcore_axis_name="s")

    @functools.partial(
        pl.kernel,
        mesh=mesh,
        out_type=jax.ShapeDtypeStruct((batch, hidden), jnp.float32),
        scratch_types=[
            pltpu.VMEM((n_chunks, CHUNK), jnp.int32),
            pltpu.VMEM((b_per_w, hidden), jnp.float32),
            pltpu.SemaphoreType.DMA,
        ],
        compiler_params=pltpu.CompilerParams(use_tc_tiling_on_sc=False),
    )
    def gather_kernel(table_hbm, idx_hbm, out_hbm, idx_v, rows_v, sem):
        wid = lax.axis_index("s") * info.num_cores + lax.axis_index("c")
        base = wid * b_per_w
        pltpu.sync_copy(idx_hbm.at[wid], idx_v)
        copies = [
            pltpu.async_copy(
                table_hbm.at[idx_v.at[j]],
                rows_v.at[pl.ds(j * CHUNK, CHUNK)],
                sem,
            )
            for j in range(n_chunks)
        ]
        for c in copies:
            c.wait()
        pltpu.sync_copy(rows_v, out_hbm.at[pl.ds(base, b_per_w)])

    return gather_kernel, nw, n_chunks


def kernel(labels, embedding_table):
    num_rows, hidden = embedding_table.shape
    batch = labels.shape[0]
    gather_kernel, nw, n_chunks = _build(num_rows, batch, hidden)
    idx3 = labels.astype(jnp.int32).reshape(nw, n_chunks, CHUNK)
    return gather_kernel(embedding_table, idx3)


# speedup vs baseline: 2.8515x; 2.8515x over previous
import functools

import jax
import jax.numpy as jnp
from jax import lax
from jax.experimental import pallas as pl
from jax.experimental.pallas import tpu as pltpu
from jax.experimental.pallas import tpu_sc as plsc

L = 16
NBUF = 8
WPF = 4
CHUNK = WPF * L


@functools.cache
def _build(hidden: int, num_rows: int, batch: int):
    info = plsc.get_sparse_core_info()
    nc = info.num_cores
    nw = nc * info.num_subcores
    b_per_w = batch // nw
    n_flush = b_per_w // CHUNK

    mesh = plsc.VectorSubcoreMesh(core_axis_name="c", subcore_axis_name="s")

    scratch = {
        "idx_v": pltpu.VMEM((b_per_w,), jnp.int32),
        "rows_v": pltpu.VMEM((CHUNK, 128), jnp.float32),
        "tiles_v": [pltpu.VMEM((hidden, 128), jnp.float32) for _ in range(NBUF)],
        "sems": [pltpu.SemaphoreType.DMA for _ in range(NBUF)],
    }

    @functools.partial(
        pl.kernel,
        mesh=mesh,
        out_type=jax.ShapeDtypeStruct((batch, 128), jnp.float32),
        scratch_types=scratch,
        compiler_params=pltpu.CompilerParams(needs_layout_passes=False),
    )
    def gather_kernel(table_hbm, idx_hbm, out_hbm, idx_v, rows_v, tiles_v, sems):
        wid = lax.axis_index("s") * nc + lax.axis_index("c")
        base = wid * b_per_w
        pltpu.sync_copy(idx_hbm.at[pl.ds(base, b_per_w)], idx_v)

        row_idx = [lax.iota(jnp.int32, L) + q * L for q in range(hidden // L)]

        def fire(r, slot):
            off = pl.multiple_of((r >> 7) * 128, 128)
            return pltpu.async_copy(
                table_hbm.at[:, pl.ds(off, 128)], tiles_v[slot], sems[slot]
            )

        def extract(r, slot, pos):
            col = jnp.full((L,), r & 127, jnp.int32)
            dst_row = jnp.full((L,), pos, jnp.int32)
            for q in range(hidden // L):
                vals = plsc.load_gather(tiles_v[slot], [row_idx[q], col])
                plsc.store_scatter(rows_v, [dst_row, row_idx[q]], vals)

        def flush_body(f, _):
            vecs = [idx_v[pl.ds((f * WPF + w) * L, L)] for w in range(WPF)]
            rs = [vecs[w][j] for w in range(WPF) for j in range(L)]
            copies = [fire(rs[k], k) for k in range(NBUF)]
            for k in range(CHUNK):
                copies[k % NBUF].wait()
                extract(rs[k], k % NBUF, k)
                if k + NBUF < CHUNK:
                    copies[(k + NBUF) % NBUF] = fire(rs[k + NBUF], k % NBUF)
            pltpu.sync_copy(
                rows_v, out_hbm.at[pl.ds(base + f * CHUNK, CHUNK)]
            )
            return 0

        lax.fori_loop(0, n_flush, flush_body, 0)

    return gather_kernel


def kernel(labels, embedding_table):
    num_rows, hidden = embedding_table.shape
    batch = labels.shape[0]
    gather_kernel = _build(hidden, num_rows, batch)
    out128 = gather_kernel(embedding_table.T, labels.astype(jnp.int32))
    return out128[:, :hidden]


# speedup vs baseline: 2.9136x; 1.0218x over previous
import functools

import jax
import jax.numpy as jnp
from jax import lax
from jax.experimental import pallas as pl
from jax.experimental.pallas import tpu as pltpu
from jax.experimental.pallas import tpu_sc as plsc

L = 16
NBUF = 8
WPF = 4
CHUNK = WPF * L


@functools.cache
def _build(hidden: int, num_rows: int, batch: int):
    info = plsc.get_sparse_core_info()
    nc = info.num_cores
    nw = nc * info.num_subcores
    b_per_w = batch // nw
    n_flush = b_per_w // CHUNK

    mesh = plsc.VectorSubcoreMesh(core_axis_name="c", subcore_axis_name="s")

    scratch = {
        "idx_v": pltpu.VMEM((b_per_w,), jnp.int32),
        "rows_v": [pltpu.VMEM((CHUNK, 128), jnp.float32) for _ in range(2)],
        "tiles_v": [pltpu.VMEM((hidden, 128), jnp.float32) for _ in range(NBUF)],
        "sems": [pltpu.SemaphoreType.DMA for _ in range(NBUF)],
        "out_sem": pltpu.SemaphoreType.DMA,
    }

    n_pair = n_flush // 2
    PAIR = 2 * CHUNK

    @functools.partial(
        pl.kernel,
        mesh=mesh,
        out_type=jax.ShapeDtypeStruct((batch, 128), jnp.float32),
        scratch_types=scratch,
        compiler_params=pltpu.CompilerParams(needs_layout_passes=False),
    )
    def gather_kernel(table_hbm, idx_hbm, out_hbm, idx_v, rows_v, tiles_v,
                      sems, out_sem):
        wid = lax.axis_index("s") * nc + lax.axis_index("c")
        base = wid * b_per_w
        pltpu.sync_copy(idx_hbm.at[pl.ds(base, b_per_w)], idx_v)

        row_idx = [lax.iota(jnp.int32, L) + q * L for q in range(hidden // L)]

        def fire(r, slot):
            off = pl.multiple_of((r >> 7) * 128, 128)
            return pltpu.async_copy(
                table_hbm.at[:, pl.ds(off, 128)], tiles_v[slot], sems[slot]
            )

        def extract(r, slot, buf, pos):
            col = jnp.full((L,), r & 127, jnp.int32)
            dst_row = jnp.full((L,), pos, jnp.int32)
            for q in range(hidden // L):
                vals = plsc.load_gather(tiles_v[slot], [row_idx[q], col])
                plsc.store_scatter(rows_v[buf], [dst_row, row_idx[q]], vals)

        def drain_out(buf):
            pltpu.make_async_copy(
                table_hbm.at[:CHUNK, pl.ds(0, 128)], rows_v[buf], out_sem
            ).wait()

        vec0 = idx_v[pl.ds(0, L)]
        prime = [fire(vec0[j], j) for j in range(NBUF)]
        for c in prime:
            del c

        def pair_body(g, _):
            gbase = g * PAIR
            vecs = [idx_v[pl.ds(gbase + w * L, L)] for w in range(PAIR // L)]
            la_off = jnp.minimum(gbase + PAIR, b_per_w - L)
            vecs.append(idx_v[pl.ds(la_off, L)])
            rs = [vecs[w][j] for w in range(len(vecs)) for j in range(L)]
            for buf in range(2):
                @pl.when(g > 0)
                def _():
                    drain_out(buf)
                for k in range(CHUNK):
                    kk = buf * CHUNK + k
                    slot = kk % NBUF
                    pltpu.make_async_copy(
                        table_hbm.at[:, pl.ds(0, 128)], tiles_v[slot],
                        sems[slot],
                    ).wait()
                    extract(rs[kk], slot, buf, k)
                    if kk + NBUF < PAIR:
                        fire(rs[kk + NBUF], slot)
                    else:
                        @pl.when(g < n_pair - 1)
                        def _():
                            fire(rs[kk + NBUF], slot)
                pltpu.async_copy(
                    rows_v[buf],
                    out_hbm.at[pl.ds(base + gbase + buf * CHUNK, CHUNK)],
                    out_sem,
                )
            return 0

        lax.fori_loop(0, n_pair, pair_body, 0)
        drain_out(0)
        drain_out(1)

    return gather_kernel


def kernel(labels, embedding_table):
    num_rows, hidden = embedding_table.shape
    batch = labels.shape[0]
    gather_kernel = _build(hidden, num_rows, batch)
    out128 = gather_kernel(embedding_table.T, labels.astype(jnp.int32))
    return out128[:, :hidden]
